# baseline (speedup 1.0000x reference)
import functools

import jax
import jax.numpy as jnp
from jax import lax
from jax.experimental import pallas as pl
from jax.experimental.pallas import tpu as pltpu
from jax.experimental.pallas import tpu_sc as plsc

_N = 10000
_E = 320000
_D = 128
_H = 128
_C = 64

_NC = 2
_NS = 16
_NW = _NC * _NS
_K = 128
_ECH0 = 80
_ECH1 = 80
_ECHM = max(_ECH0, _ECH1)
_E_CAP = (_ECH0 + _ECH1) * _NS * _K
_DCH = 79
_E_PAD = _DCH * _NW * _K
_RPT = 632
_N_PAD = _RPT * _NS
_BR = 2000


def _deg_body(dst_hbm, ones_hbm, zeros_hbm, out_hbm, dst_v, ones_v, acc):
    cid = lax.axis_index("c")
    sid = lax.axis_index("s")
    wid = cid * _NS + sid
    pltpu.sync_copy(zeros_hbm, acc.at[pl.ds(sid * _RPT, _RPT)])
    pltpu.sync_copy(dst_hbm.at[wid], dst_v)
    pltpu.sync_copy(ones_hbm, ones_v)
    plsc.subcore_barrier()

    def body(c, carry):
        pltpu.sync_copy(ones_v, acc.at[dst_v.at[c]], add=True)
        return carry

    lax.fori_loop(0, _DCH, body, 0)
    plsc.subcore_barrier()
    pltpu.sync_copy(acc.at[pl.ds(sid * _RPT, _RPT)],
                    out_hbm.at[cid, pl.ds(sid * _RPT, _RPT)])


_deg_call = pl.kernel(
    _deg_body,
    out_type=jax.ShapeDtypeStruct((_NC, _N_PAD, _D), jnp.float32),
    mesh=plsc.VectorSubcoreMesh(core_axis_name="c", subcore_axis_name="s"),
    scratch_types=[
        pltpu.VMEM((_DCH, _K), jnp.int32),
        pltpu.VMEM((_K, _D), jnp.float32),
        pltpu.VMEM_SHARED((_N_PAD, _D), jnp.float32),
    ],
)


def _msg_body(h_hbm, edges_hbm, zeros_hbm, out_hbm,
              ed0, ed1, ed2, ed3, rows0, rows1, acc,
              ie0, ie1, ie2, ie3, rs0, rs1):
    cid = lax.axis_index("c")
    sid = lax.axis_index("s")
    wid = cid * _NS + sid
    nch = jnp.where(cid == 0, _ECH0, _ECH1)
    ed = (ed0, ed1, ed2, ed3)
    ie = (ie0, ie1, ie2, ie3)
    rows = (rows0, rows1)
    rs = (rs0, rs1)

    def chw(c):
        return jnp.where(c < nch, c, 0)

    pltpu.sync_copy(zeros_hbm, acc.at[pl.ds(sid * _RPT, _RPT)])
    plsc.subcore_barrier()

    for j in range(4):
        pltpu.async_copy(edges_hbm.at[wid, chw(j)], ed[j], ie[j])
    for j in range(2):
        pltpu.make_async_copy(edges_hbm.at[wid, 0], ed[j], ie[j]).wait()
        pltpu.async_copy(h_hbm.at[ed[j].at[0]], rows[j], rs[j])

    def body(q, carry):
        for i in range(4):
            c = q * 4 + i
            pltpu.make_async_copy(h_hbm.at[ed[i].at[0]], rows[i % 2],
                                  rs[i % 2]).wait()
            pltpu.sync_copy(rows[i % 2], acc.at[ed[i].at[1]], add=True)
            pltpu.async_copy(edges_hbm.at[wid, chw(c + 4)], ed[i], ie[i])
            pltpu.make_async_copy(edges_hbm.at[wid, 0], ed[(i + 2) % 4],
                                  ie[(i + 2) % 4]).wait()
            pltpu.async_copy(h_hbm.at[ed[(i + 2) % 4].at[0]],
                             rows[i % 2], rs[i % 2])
        return carry

    lax.fori_loop(0, nch // 4, body, 0)
    for j in range(2):
        pltpu.make_async_copy(h_hbm.at[ed[j].at[0]], rows[j], rs[j]).wait()
        pltpu.make_async_copy(edges_hbm.at[wid, 0], ed[j + 2],
                              ie[j + 2]).wait()
    plsc.subcore_barrier()
    pltpu.sync_copy(acc.at[pl.ds(sid * _RPT, _RPT)],
                    out_hbm.at[cid, pl.ds(sid * _RPT, _RPT)])


_msg_call = pl.kernel(
    _msg_body,
    out_type=jax.ShapeDtypeStruct((_NC, _N_PAD, _D), jnp.float32),
    mesh=plsc.VectorSubcoreMesh(core_axis_name="c", subcore_axis_name="s"),
    scratch_types=[
        pltpu.VMEM((2, _K), jnp.int32),
        pltpu.VMEM((2, _K), jnp.int32),
        pltpu.VMEM((2, _K), jnp.int32),
        pltpu.VMEM((2, _K), jnp.int32),
        pltpu.VMEM((_K, _D), jnp.float32),
        pltpu.VMEM((_K, _D), jnp.float32),
        pltpu.VMEM_SHARED((_N_PAD, _D), jnp.float32),
        pltpu.SemaphoreType.DMA,
        pltpu.SemaphoreType.DMA,
        pltpu.SemaphoreType.DMA,
        pltpu.SemaphoreType.DMA,
        pltpu.SemaphoreType.DMA,
        pltpu.SemaphoreType.DMA,
    ],
)


def _prep_body(degp_ref, x_ref, normb_ref, h0_ref):
    p = degp_ref[...]
    deg = p[0, :, 0] + p[1, :, 0]
    norm = lax.rsqrt(jnp.maximum(deg, 1.0))
    nb = jnp.broadcast_to(norm[:, None], (_BR, _D))
    normb_ref[...] = nb
    h0_ref[...] = x_ref[...] * nb


def _prep_call(degp, x):
    return pl.pallas_call(
        _prep_body,
        grid=(_N // _BR,),
        in_specs=[
            pl.BlockSpec((_NC, _BR, _D), lambda i: (0, i, 0)),
            pl.BlockSpec((_BR, _D), lambda i: (i, 0)),
        ],
        out_specs=[
            pl.BlockSpec((_BR, _D), lambda i: (i, 0)),
            pl.BlockSpec((_BR, _D), lambda i: (i, 0)),
        ],
        out_shape=[
            jax.ShapeDtypeStruct((_N, _D), jnp.float32),
            jax.ShapeDtypeStruct((_N, _D), jnp.float32),
        ],
    )(degp, x)


def _layer_body(mp_ref, normb_ref, w_ref, b_ref, out_ref, *, last):
    p = mp_ref[...]
    m = p[0] + p[1]
    h = m * normb_ref[...]
    y = jnp.dot(h, w_ref[...], preferred_element_type=jnp.float32)
    y = y + b_ref[...][None, :]
    if last:
        out_ref[...] = y
    else:
        out_ref[...] = jnp.maximum(y, 0.0) * normb_ref[...]


def _layer_call(mp, normb, w, b, last):
    wout = w.shape[1]
    return pl.pallas_call(
        functools.partial(_layer_body, last=last),
        grid=(_N // _BR,),
        in_specs=[
            pl.BlockSpec((_NC, _BR, _D), lambda i: (0, i, 0)),
            pl.BlockSpec((_BR, _D), lambda i: (i, 0)),
            pl.BlockSpec((_D, wout), lambda i: (0, 0)),
            pl.BlockSpec((wout,), lambda i: (0,)),
        ],
        out_specs=pl.BlockSpec((_BR, wout), lambda i: (i, 0)),
        out_shape=jax.ShapeDtypeStruct((_N, wout), jnp.float32),
    )(mp, normb, w, b)


def _chunk_split(v, fill):
    a0 = _NS * _ECH0 * _K
    a1 = _NS * _ECH1 * _K
    vp = jnp.concatenate([v, jnp.full((a0 + a1 - _E,), fill, jnp.int32)])
    v0 = vp[:a0].reshape(_NS, _ECH0, _K)
    v1 = vp[a0:].reshape(_NS, _ECH1, _K)
    if _ECH0 < _ECHM:
        v0 = jnp.concatenate(
            [v0, jnp.full((_NS, _ECHM - _ECH0, _K), fill, jnp.int32)], 1)
    if _ECH1 < _ECHM:
        v1 = jnp.concatenate(
            [v1, jnp.full((_NS, _ECHM - _ECH1, _K), fill, jnp.int32)], 1)
    return jnp.concatenate([v0, v1], 0)


def kernel(x, edge_index, W0, b0, W1, b1, W2, b2, W3, b3):
    src = edge_index[0]
    dst = edge_index[1]
    srcc = _chunk_split(src, 0)
    dstc = _chunk_split(dst, _N)
    edges = jnp.stack([srcc, dstc], axis=2)
    padd = _E_PAD - _E
    dst3 = jnp.concatenate([dst, jnp.full((padd,), _N, jnp.int32)]).reshape(
        _NW, _DCH, _K)
    zeros_m = jnp.zeros((_RPT, _D), jnp.float32)
    ones_m = jnp.ones((_K, _D), jnp.float32)

    degp = _deg_call(dst3, ones_m, zeros_m)
    normb, h = _prep_call(degp, x)
    for w, b, last in ((W0, b0, False), (W1, b1, False),
                      (W2, b2, False), (W3, b3, True)):
        mp = _msg_call(h, edges, zeros_m)
        h = _layer_call(mp, normb, w, b, last)
    return h
